# Initial kernel scaffold; baseline (speedup 1.0000x reference)
import jax
import jax.numpy as jnp
from jax.experimental import pallas as pl


def kernel(coord_idx, position):
    raise NotImplementedError("write your pallas kernel here")


# speedup vs baseline: 4.5831x; 4.5831x over previous
import functools

import jax
import jax.numpy as jnp
from jax import lax
from jax.experimental import pallas as pl
from jax.experimental.pallas import tpu as pltpu
from jax.experimental.pallas import tpu_sc as plsc

EMBED = 192
TABLE_ROWS = 64 * 32
N = 128 * 2048
NC, NS, L = 2, 16, 16
NW = NC * NS
B_PER_W = N // NW
CHUNK = 128
ITERS = B_PER_W // CHUNK

_mesh = plsc.VectorSubcoreMesh(core_axis_name="c", subcore_axis_name="s")


_DNUMS = lax.GatherDimensionNumbers(
    offset_dims=(), collapsed_slice_dims=(0,), start_index_map=(0,))


def _take(v, idx):
    return lax.gather(v, idx[:, None], _DNUMS, (1,),
                      mode=lax.GatherScatterMode.PROMISE_IN_BOUNDS)


@functools.partial(
    pl.kernel,
    out_type=jax.ShapeDtypeStruct((N, EMBED), jnp.float32),
    mesh=_mesh,
    compiler_params=pltpu.CompilerParams(use_tc_tiling_on_sc=False),
    scratch_types=[
        pltpu.VMEM((2 * CHUNK,), jnp.int32),
        pltpu.VMEM((CHUNK,), jnp.int32),
        pltpu.VMEM((CHUNK, EMBED), jnp.float32),
        pltpu.SemaphoreType.DMA,
    ],
)
def _gather(coord_hbm, table_hbm, out_hbm, coords_v, idx_v, rows_v, sem):
    wid = lax.axis_index("s") * NC + lax.axis_index("c")
    base0 = wid * B_PER_W
    lanes = lax.iota(jnp.int32, L)
    evens = (lanes * 2) % L
    lo = lanes < (L // 2)

    def chunk(it, carry):
        base = base0 + it * CHUNK
        pltpu.sync_copy(coord_hbm.at[pl.ds(base * 2, 2 * CHUNK)], coords_v)
        for i in range(CHUNK // L):
            a = coords_v[pl.ds(2 * L * i, L)]
            b = coords_v[pl.ds(2 * L * i + L, L)]
            c0 = jnp.where(lo, _take(a, evens), _take(b, evens))
            c1 = jnp.where(lo, _take(a, evens + 1), _take(b, evens + 1))
            idx_v[pl.ds(i * L, L)] = c0 * 32 + c1
        pltpu.async_copy(table_hbm.at[idx_v], rows_v, sem).wait()
        pltpu.sync_copy(rows_v, out_hbm.at[pl.ds(base, CHUNK)])
        return carry

    lax.fori_loop(0, ITERS, chunk, 0)


def kernel(coord_idx, position):
    coords = coord_idx.reshape(-1)
    table = position.reshape(TABLE_ROWS, EMBED)
    return _gather(coords, table)


# speedup vs baseline: 4.8672x; 1.0620x over previous
import functools

import jax
import jax.numpy as jnp
from jax import lax
from jax.experimental import pallas as pl
from jax.experimental.pallas import tpu as pltpu
from jax.experimental.pallas import tpu_sc as plsc

EMBED = 192
TABLE_ROWS = 64 * 32
N = 128 * 2048
NC, NS, L = 2, 16, 16
NW = NC * NS
B_PER_W = N // NW
CHUNK = 256
NIDX = CHUNK // 128
GROUPS = CHUNK // L
ITERS = B_PER_W // CHUNK

_mesh = plsc.VectorSubcoreMesh(core_axis_name="c", subcore_axis_name="s")

_DNUMS = lax.GatherDimensionNumbers(
    offset_dims=(), collapsed_slice_dims=(0,), start_index_map=(0,))


def _take(v, idx):
    return lax.gather(v, idx[:, None], _DNUMS, (1,),
                      mode=lax.GatherScatterMode.PROMISE_IN_BOUNDS)


@functools.partial(
    pl.kernel,
    out_type=jax.ShapeDtypeStruct((N, EMBED), jnp.float32),
    mesh=_mesh,
    compiler_params=pltpu.CompilerParams(use_tc_tiling_on_sc=False),
    scratch_types=[
        pltpu.VMEM((2, 2 * CHUNK), jnp.int32),
        pltpu.VMEM((2, NIDX, 128), jnp.int32),
        pltpu.VMEM((2, CHUNK, EMBED), jnp.float32),
        pltpu.SemaphoreType.DMA,
        pltpu.SemaphoreType.DMA,
        pltpu.SemaphoreType.DMA,
        pltpu.SemaphoreType.DMA,
        pltpu.SemaphoreType.DMA,
    ],
)
def _gather(coord_hbm, table_hbm, out_hbm, coords_v, idx_v, rows_v,
            sem_c0, sem_c1, sem_g, sem_o0, sem_o1):
    wid = lax.axis_index("s") * NC + lax.axis_index("c")
    base0 = wid * B_PER_W
    lanes = lax.iota(jnp.int32, L)
    evens = (lanes * 2) % L
    lo = lanes < (L // 2)
    sem_c = (sem_c0, sem_c1)
    sem_o = (sem_o0, sem_o1)

    def coords_copy(i, b):
        base = base0 + i * CHUNK
        return pltpu.make_async_copy(
            coord_hbm.at[pl.ds(base * 2, 2 * CHUNK)], coords_v.at[b], sem_c[b])

    def out_copy(i, b):
        base = base0 + i * CHUNK
        return pltpu.make_async_copy(
            rows_v.at[b], out_hbm.at[pl.ds(base, CHUNK)], sem_o[b])

    def step(i, b, first):
        coords_copy(i, b).wait()
        for g in range(GROUPS):
            a = coords_v[b, pl.ds(2 * L * g, L)]
            bb = coords_v[b, pl.ds(2 * L * g + L, L)]
            c0 = jnp.where(lo, _take(a, evens), _take(bb, evens))
            c1 = jnp.where(lo, _take(a, evens + 1), _take(bb, evens + 1))
            idx_v[b, g // 8, pl.ds((g % 8) * L, L)] = c0 * 32 + c1
        nxt = jnp.minimum(i + 2, ITERS - 1)
        coords_copy(nxt, b).start()
        if not first:
            out_copy(i - 2, b).wait()
        gs = [pltpu.async_copy(table_hbm.at[idx_v.at[b, j]],
                               rows_v.at[b, pl.ds(128 * j, 128)], sem_g)
              for j in range(NIDX)]
        for h in gs:
            h.wait()
        out_copy(i, b).start()

    coords_copy(0, 0).start()
    coords_copy(1, 1).start()
    step(0, 0, True)
    step(1, 1, True)

    def pair(it2, carry):
        i = it2 * 2
        step(i, 0, False)
        step(i + 1, 1, False)
        return carry

    lax.fori_loop(1, ITERS // 2, pair, 0)

    for b, i_last in ((0, ITERS - 2), (1, ITERS - 1)):
        out_copy(i_last, b).wait()
        coords_copy(ITERS - 1, b).wait()


def kernel(coord_idx, position):
    coords = coord_idx.reshape(-1)
    table = position.reshape(TABLE_ROWS, EMBED)
    return _gather(coords, table)


# speedup vs baseline: 5.6609x; 1.1631x over previous
import functools

import jax
import jax.numpy as jnp
from jax import lax
from jax.experimental import pallas as pl
from jax.experimental.pallas import tpu as pltpu
from jax.experimental.pallas import tpu_sc as plsc

EMBED = 192
TABLE_ROWS = 64 * 32
N = 128 * 2048
NC, NS, L = 2, 16, 16
NW = NC * NS
B_PER_W = N // NW
CHUNK = 256
NIDX = CHUNK // 128
GROUPS = CHUNK // L
ITERS = B_PER_W // CHUNK

_mesh = plsc.VectorSubcoreMesh(core_axis_name="c", subcore_axis_name="s")

_DNUMS = lax.GatherDimensionNumbers(
    offset_dims=(), collapsed_slice_dims=(0,), start_index_map=(0,))


def _take(v, idx):
    return lax.gather(v, idx[:, None], _DNUMS, (1,),
                      mode=lax.GatherScatterMode.PROMISE_IN_BOUNDS)


@functools.partial(
    pl.kernel,
    out_type=jax.ShapeDtypeStruct((N, EMBED), jnp.float32),
    mesh=_mesh,
    compiler_params=pltpu.CompilerParams(use_tc_tiling_on_sc=False),
    scratch_types=[
        pltpu.VMEM((2, 2 * CHUNK), jnp.int32),
        pltpu.VMEM((2, NIDX, 128), jnp.int32),
        pltpu.VMEM((2, CHUNK, EMBED), jnp.float32),
        pltpu.VMEM_SHARED((TABLE_ROWS, EMBED), jnp.float32),
        pltpu.SemaphoreType.DMA,
        pltpu.SemaphoreType.DMA,
        pltpu.SemaphoreType.DMA,
        pltpu.SemaphoreType.DMA,
        pltpu.SemaphoreType.DMA,
    ],
)
def _gather(coord_hbm, table_hbm, out_hbm, coords_v, idx_v, rows_v,
            table_sp, sem_c0, sem_c1, sem_g, sem_o0, sem_o1):
    sid = lax.axis_index("s")
    wid = sid * NC + lax.axis_index("c")
    base0 = wid * B_PER_W

    slab = TABLE_ROWS // NS
    pltpu.sync_copy(table_hbm.at[pl.ds(sid * slab, slab)],
                    rows_v.at[0, pl.ds(0, slab)])
    pltpu.sync_copy(rows_v.at[0, pl.ds(0, slab)],
                    table_sp.at[pl.ds(sid * slab, slab)])
    plsc.subcore_barrier()
    lanes = lax.iota(jnp.int32, L)
    evens = (lanes * 2) % L
    lo = lanes < (L // 2)
    sem_c = (sem_c0, sem_c1)
    sem_o = (sem_o0, sem_o1)

    def coords_copy(i, b):
        base = base0 + i * CHUNK
        return pltpu.make_async_copy(
            coord_hbm.at[pl.ds(base * 2, 2 * CHUNK)], coords_v.at[b], sem_c[b])

    def out_copy(i, b):
        base = base0 + i * CHUNK
        return pltpu.make_async_copy(
            rows_v.at[b], out_hbm.at[pl.ds(base, CHUNK)], sem_o[b])

    def step(i, b, first):
        coords_copy(i, b).wait()
        for g in range(GROUPS):
            a = coords_v[b, pl.ds(2 * L * g, L)]
            bb = coords_v[b, pl.ds(2 * L * g + L, L)]
            c0 = jnp.where(lo, _take(a, evens), _take(bb, evens))
            c1 = jnp.where(lo, _take(a, evens + 1), _take(bb, evens + 1))
            idx_v[b, g // 8, pl.ds((g % 8) * L, L)] = c0 * 32 + c1
        nxt = jnp.minimum(i + 2, ITERS - 1)
        coords_copy(nxt, b).start()
        if not first:
            out_copy(i - 2, b).wait()
        gs = [pltpu.async_copy(table_sp.at[idx_v.at[b, j]],
                               rows_v.at[b, pl.ds(128 * j, 128)], sem_g)
              for j in range(NIDX)]
        for h in gs:
            h.wait()
        out_copy(i, b).start()

    coords_copy(0, 0).start()
    coords_copy(1, 1).start()
    step(0, 0, True)
    step(1, 1, True)

    def pair(it2, carry):
        i = it2 * 2
        step(i, 0, False)
        step(i + 1, 1, False)
        return carry

    lax.fori_loop(1, ITERS // 2, pair, 0)

    for b, i_last in ((0, ITERS - 2), (1, ITERS - 1)):
        out_copy(i_last, b).wait()
        coords_copy(ITERS - 1, b).wait()


def kernel(coord_idx, position):
    coords = coord_idx.reshape(-1)
    table = position.reshape(TABLE_ROWS, EMBED)
    return _gather(coords, table)


# speedup vs baseline: 5.6629x; 1.0004x over previous
import functools

import jax
import jax.numpy as jnp
from jax import lax
from jax.experimental import pallas as pl
from jax.experimental.pallas import tpu as pltpu
from jax.experimental.pallas import tpu_sc as plsc

EMBED = 192
TABLE_ROWS = 64 * 32
N = 128 * 2048
NC, NS, L = 2, 16, 16
NW = NC * NS
B_PER_W = N // NW
BLK = 128
NBLK = B_PER_W // BLK
NRING = 3
DEPTH = 2

_mesh = plsc.VectorSubcoreMesh(core_axis_name="c", subcore_axis_name="s")

_DNUMS = lax.GatherDimensionNumbers(
    offset_dims=(), collapsed_slice_dims=(0,), start_index_map=(0,))


def _take(v, idx):
    return lax.gather(v, idx[:, None], _DNUMS, (1,),
                      mode=lax.GatherScatterMode.PROMISE_IN_BOUNDS)


@functools.partial(
    pl.kernel,
    out_type=jax.ShapeDtypeStruct((N, EMBED), jnp.float32),
    mesh=_mesh,
    compiler_params=pltpu.CompilerParams(use_tc_tiling_on_sc=False),
    scratch_types=[
        pltpu.VMEM((2 * B_PER_W,), jnp.int32),
        pltpu.VMEM((NBLK, BLK), jnp.int32),
        pltpu.VMEM((NRING, BLK, EMBED), jnp.float32),
        pltpu.VMEM_SHARED((TABLE_ROWS, EMBED), jnp.float32),
        pltpu.SemaphoreType.DMA,
        pltpu.SemaphoreType.DMA,
        pltpu.SemaphoreType.DMA,
        pltpu.SemaphoreType.DMA,
        pltpu.SemaphoreType.DMA,
        pltpu.SemaphoreType.DMA,
        pltpu.SemaphoreType.DMA,
    ],
)
def _gather(coord_hbm, table_hbm, out_hbm, coords_v, idx_v, rows_v, table_sp,
            sem_c, sg0, sg1, sg2, so0, so1, so2):
    sem_g = (sg0, sg1, sg2)
    sem_o = (so0, so1, so2)
    sid = lax.axis_index("s")
    wid = sid * NC + lax.axis_index("c")
    base0 = wid * B_PER_W

    ccopy = pltpu.make_async_copy(
        coord_hbm.at[pl.ds(base0 * 2, 2 * B_PER_W)], coords_v, sem_c)
    ccopy.start()
    slab = TABLE_ROWS // NS
    pltpu.sync_copy(table_hbm.at[pl.ds(sid * slab, slab)],
                    rows_v.at[0, pl.ds(0, slab)])
    pltpu.sync_copy(rows_v.at[0, pl.ds(0, slab)],
                    table_sp.at[pl.ds(sid * slab, slab)])
    plsc.subcore_barrier()
    ccopy.wait()

    lanes = lax.iota(jnp.int32, L)
    evens = (lanes * 2) % L
    lo = lanes < (L // 2)

    def degroup(it, carry):
        for u in range(8):
            g = it * 8 + u
            a = coords_v[pl.ds(2 * L * g, L)]
            b = coords_v[pl.ds(2 * L * g + L, L)]
            c0 = jnp.where(lo, _take(a, evens), _take(b, evens))
            c1 = jnp.where(lo, _take(a, evens + 1), _take(b, evens + 1))
            idx_v[it, pl.ds(u * L, L)] = c0 * 32 + c1
        return carry

    lax.fori_loop(0, (B_PER_W // L) // 8, degroup, 0)

    def gat(i, b):
        return pltpu.make_async_copy(
            table_sp.at[idx_v.at[i]], rows_v.at[b], sem_g[b])

    def put(i, b):
        return pltpu.make_async_copy(
            rows_v.at[b], out_hbm.at[pl.ds(base0 + i * BLK, BLK)], sem_o[b])

    def step(i, im, first, last):
        bwf = (im + DEPTH) % NRING
        if not first:
            put(i + DEPTH - NRING, bwf).wait()
        if not last:
            gat(i + DEPTH, bwf).start()
        gat(i, im).wait()
        put(i, im).start()

    for i in range(DEPTH):
        gat(i, i % NRING).start()
    step(0, 0, True, False)
    step(1, 1, False, False)

    def triple(it, carry):
        i0 = 2 + it * NRING
        for u in range(NRING):
            step(i0 + u, (2 + u) % NRING, False, False)
        return carry

    lax.fori_loop(0, (NBLK - DEPTH - 2) // NRING, triple, 0)

    for i in range(NBLK - DEPTH, NBLK):
        step(i, i % NRING, False, True)
    put(NBLK - 1, (NBLK - 1) % NRING).wait()


def kernel(coord_idx, position):
    coords = coord_idx.reshape(-1)
    table = position.reshape(TABLE_ROWS, EMBED)
    return _gather(coords, table)
